# Initial kernel scaffold; baseline (speedup 1.0000x reference)
import jax
import jax.numpy as jnp
from jax.experimental import pallas as pl


def kernel(x, q_raw, alpha_raw):
    raise NotImplementedError("write your pallas kernel here")


# speedup vs baseline: 6.8541x; 6.8541x over previous
import jax
import jax.numpy as jnp
from jax.experimental import pallas as pl
from jax.experimental.pallas import tpu as pltpu

_B, _C, _L = 16, 128, 16384
_K, _S = 8, 4
_ITERS = 20
_W = (_L - _K) // _S + 1
_WP = 4096
_WC = 512
_NW = _WP // _WC
_LP = _S * _WP + _K


def _pool_kernel(q_ref, a_ref, x_ref, o_ref):
    wi = pl.program_id(1)
    base = wi * (_S * _WC)
    alpha = jnp.exp(a_ref[...])
    q8 = _K * jax.nn.sigmoid(q_ref[...])
    inv_alpha = jnp.exp(-a_ref[...])

    xs = [alpha * x_ref[0, pl.ds(base + k, _WC, _S), :] for k in range(_K)]

    mn = xs[0]
    mx = xs[0]
    for t in xs[1:]:
        mn = jnp.minimum(mn, t)
        mx = jnp.maximum(mx, t)
    lo = mn - 2.0
    hi = mx + 2.0

    ps = [jnp.exp(t) for t in xs]

    def body(_, carry):
        lo, hi = carry
        c = 0.5 * (lo + hi)
        u = jnp.exp(c)
        acc = 1.0 / (u + ps[0])
        for p in ps[1:]:
            acc = acc + 1.0 / (u + p)
        s = u * acc
        th = s > q8
        return (jnp.where(th, lo, c), jnp.where(th, c, hi))

    lo, hi = jax.lax.fori_loop(0, _ITERS, body, (lo, hi))
    o_ref[0] = (0.5 * (lo + hi)) * inv_alpha


@jax.jit
def kernel(x, q_raw, alpha_raw):
    xt = jnp.transpose(x, (0, 2, 1))
    xt = jnp.pad(xt, ((0, 0), (0, _LP - _L), (0, 0)))
    q2 = q_raw.reshape(1, _C)
    a2 = alpha_raw.reshape(1, _C)
    out = pl.pallas_call(
        _pool_kernel,
        grid=(_B, _NW),
        in_specs=[
            pl.BlockSpec((1, _C), lambda b, w: (0, 0)),
            pl.BlockSpec((1, _C), lambda b, w: (0, 0)),
            pl.BlockSpec((1, _LP, _C), lambda b, w: (b, 0, 0)),
        ],
        out_specs=pl.BlockSpec((1, _WC, _C), lambda b, w: (b, w, 0)),
        out_shape=jax.ShapeDtypeStruct((_B, _WP, _C), jnp.float32),
        compiler_params=pltpu.CompilerParams(
            dimension_semantics=("parallel", "arbitrary"),
            vmem_limit_bytes=48 * 1024 * 1024,
        ),
    )(q2, a2, xt)
    return out[:, :_W, :].transpose(0, 2, 1)
